# baseline (speedup 1.0000x reference)
import jax
import jax.numpy as jnp
from jax import lax
from jax.experimental import pallas as pl
from jax.experimental.pallas import tpu as pltpu


def _body(tt_ref, x_ref, w8_ref, o_ref):
    i = pl.program_id(0)
    bn = x_ref.shape[0]
    tm = tt_ref[0]
    c8 = w8_ref[pl.ds(i * bn, bn), :]
    lane = jax.lax.broadcasted_iota(jnp.int32, c8.shape, 1)
    col = jnp.sum(jnp.where(lane == tm, c8, 0.0), axis=1, keepdims=True)
    o_ref[...] = x_ref[...] - col


def kernel(x, weight, token_type):
    n, d = x.shape
    bn = 2048
    t = jnp.asarray(token_type, jnp.int32)
    t0 = (t // 8) * 8
    w8 = lax.dynamic_slice(weight, (jnp.int32(0), t0), (n, 8))
    tm = (t % 8).reshape(1)
    return pl.pallas_call(
        _body,
        grid=(n // bn,),
        in_specs=[
            pl.BlockSpec(memory_space=pltpu.SMEM),
            pl.BlockSpec((bn, d), lambda i: (i, 0)),
            pl.BlockSpec((n, 8), lambda i: (0, 0)),
        ],
        out_specs=pl.BlockSpec((bn, d), lambda i: (i, 0)),
        out_shape=jax.ShapeDtypeStruct((n, d), jnp.float32),
    )(tm, x, w8)


# speedup vs baseline: 1.1987x; 1.1987x over previous
import jax
import jax.numpy as jnp
from jax import lax
from jax.experimental import pallas as pl
from jax.experimental.pallas import tpu as pltpu


def _body(tt_ref, x_ref, w8_ref, o_ref):
    tm = tt_ref[0]
    c8 = w8_ref[...]
    lane = jax.lax.broadcasted_iota(jnp.int32, c8.shape, 1)
    col = jnp.sum(jnp.where(lane == tm, c8, 0.0), axis=1, keepdims=True)
    o_ref[...] = x_ref[...] - col


def kernel(x, weight, token_type):
    n, d = x.shape
    bn = 8192
    t = jnp.asarray(token_type, jnp.int32)
    t0 = (t // 8) * 8
    w8 = lax.dynamic_slice(weight, (jnp.int32(0), t0), (n, 8))
    tm = (t % 8).reshape(1)
    return pl.pallas_call(
        _body,
        grid=(n // bn,),
        in_specs=[
            pl.BlockSpec(memory_space=pltpu.SMEM),
            pl.BlockSpec((bn, d), lambda i: (i, 0)),
            pl.BlockSpec((bn, 8), lambda i: (i, 0)),
        ],
        out_specs=pl.BlockSpec((bn, d), lambda i: (i, 0)),
        out_shape=jax.ShapeDtypeStruct((n, d), jnp.float32),
    )(tm, x, w8)
